# Initial kernel scaffold; baseline (speedup 1.0000x reference)
import jax
import jax.numpy as jnp
from jax.experimental import pallas as pl


def kernel(input_ids, typed_edges, emb_table, params):
    raise NotImplementedError("write your pallas kernel here")


# speedup vs baseline: 5.8489x; 5.8489x over previous
import functools

import jax
import jax.numpy as jnp
from jax import lax
from jax.experimental import pallas as pl
from jax.experimental.pallas import tpu as pltpu
from jax.experimental.pallas import tpu_sc as plsc

N_NODES = 10000
N_EDGES = 160000
D = 128
T = 4
NC, NS = 2, 16
NW = NC * NS
NPAD = 10240
EPAD = 163840
ROWS_E = EPAD // 128
_INV_SQRT2 = 0.7071067811865476


def _recip(s):
    r = 1.0 / s
    r = r * (2.0 - s * r)
    r = r * (2.0 - s * r)
    return r

_mesh = plsc.VectorSubcoreMesh(
    core_axis_name="c", subcore_axis_name="s", num_cores=NC, num_subcores=NS
)


def _embed_call(emb_table, ids2):
    @functools.partial(
        pl.kernel,
        out_type=jax.ShapeDtypeStruct((NPAD, D), jnp.float32),
        mesh=_mesh,
        scratch_types=[
            pltpu.VMEM((8, 128), jnp.int32),
            pltpu.VMEM((128, D), jnp.float32),
            pltpu.SemaphoreType.DMA,
        ],
    )
    def k(emb_h, ids_h, out_h, ids_v, rows_v, sem):
        c = lax.axis_index("c")
        s = lax.axis_index("s")
        wid = s * NC + c

        @pl.when(wid < 10)
        def _():
            pltpu.sync_copy(ids_h.at[pl.ds(wid * 8, 8)], ids_v)

            @pl.loop(0, 8)
            def _(j):
                pltpu.async_copy(emb_h.at[ids_v.at[j]], rows_v, sem).wait()
                pltpu.sync_copy(rows_v, out_h.at[pl.ds((wid * 8 + j) * 128, 128)])

    return k(emb_table, ids2)


def _scatter_call(y, gidx, tgt2, zeros_init, split_cols):
    @functools.partial(
        pl.kernel,
        out_type=jax.ShapeDtypeStruct((2 * NPAD, 128), jnp.float32),
        mesh=_mesh,
        scratch_types=[
            pltpu.VMEM((8, 128), jnp.int32),
            pltpu.VMEM((8, 128), jnp.int32),
            pltpu.VMEM((128, 128), jnp.float32),
            pltpu.VMEM_SHARED((NPAD, 128), jnp.float32),
            pltpu.SemaphoreType.DMA,
        ],
    )
    def k(y_h, g_h, t_h, z_h, out_h, idx_v, tgt_v, rows_v, acc_sh, sem):
        c = lax.axis_index("c")
        s = lax.axis_index("s")
        stripe = NPAD // NS
        pltpu.sync_copy(z_h.at[pl.ds(s * stripe, stripe)],
                        acc_sh.at[pl.ds(s * stripe, stripe)])
        plsc.subcore_barrier()
        if split_cols:
            n_super = 10
            idx_base = c * ROWS_E + s * 80
            tgt_base = s * 80
        else:
            wid = s * NC + c
            n_super = 5
            idx_base = wid * 40
            tgt_base = wid * 40

        @pl.loop(0, n_super)
        def _(chunk):
            pltpu.sync_copy(g_h.at[pl.ds(idx_base + chunk * 8, 8)], idx_v)
            pltpu.sync_copy(t_h.at[pl.ds(tgt_base + chunk * 8, 8)], tgt_v)

            @pl.loop(0, 8)
            def _(j):
                pltpu.async_copy(y_h.at[idx_v.at[j]], rows_v, sem).wait()
                pltpu.sync_copy(rows_v, acc_sh.at[tgt_v.at[j]], add=True)

        plsc.subcore_barrier()
        pltpu.sync_copy(acc_sh.at[pl.ds(s * stripe, stripe)],
                        out_h.at[pl.ds(c * NPAD + s * stripe, stripe)])

    return k(y, gidx, tgt2, zeros_init)


_BN = 256
_NB = NPAD // _BN


def _pre_call(state_bf, a_split_bf):
    din = state_bf.shape[1]
    n_copies = a_split_bf.shape[0]

    def body(s_ref, a_ref, y_ref):
        y_ref[0, 0] = jnp.dot(s_ref[...], a_ref[0, 0],
                              preferred_element_type=jnp.float32)

    return pl.pallas_call(
        body,
        grid=(n_copies, T, _NB),
        in_specs=[
            pl.BlockSpec((_BN, din), lambda c, t, i: (i, 0)),
            pl.BlockSpec((1, 1, din, 128), lambda c, t, i: (c, t, 0, 0)),
        ],
        out_specs=pl.BlockSpec((1, 1, _BN, 128), lambda c, t, i: (c, t, i, 0)),
        out_shape=jax.ShapeDtypeStruct((n_copies, T, NPAD, 128), jnp.float32),
    )(state_bf, a_split_bf)


def _mix_call(state_bf, b_stk_bf, cnt_parts):
    din = state_bf.shape[1]

    def body(s_ref, b_ref, c_ref, o_ref):
        sv = s_ref[...]
        cnt = c_ref[0] + c_ref[1]
        acc = jnp.zeros((_BN, din), jnp.float32)
        for t in range(T):
            mm = jnp.dot(sv, b_ref[t], preferred_element_type=jnp.float32)
            acc = acc + cnt[:, 16 * t][:, None] * mm
        o_ref[...] = acc

    return pl.pallas_call(
        body,
        grid=(_NB,),
        in_specs=[
            pl.BlockSpec((_BN, din), lambda i: (i, 0)),
            pl.BlockSpec((T, din, din), lambda i: (0, 0, 0)),
            pl.BlockSpec((2, _BN, 128), lambda i: (0, i, 0)),
        ],
        out_specs=pl.BlockSpec((_BN, din), lambda i: (i, 0)),
        out_shape=jax.ShapeDtypeStruct((NPAD, din), jnp.float32),
    )(state_bf, b_stk_bf, cnt_parts)


def _post_call(parts, zbw, ln_g, ln_b, wd_t_bf, bd, split_cols, resid_in=None):
    din = zbw.shape[1]
    dout = wd_t_bf.shape[1]
    with_resid = resid_in is not None

    def body(*refs):
        if with_resid:
            p_ref, z_ref, g_ref, b_ref, w_ref, d_ref, r_ref, o_ref = refs
        else:
            p_ref, z_ref, g_ref, b_ref, w_ref, d_ref, o_ref = refs
        if split_cols:
            agg = jnp.concatenate([p_ref[0], p_ref[1]], axis=1)
        else:
            agg = p_ref[0] + p_ref[1]
        agg = agg + z_ref[...]
        h = (agg * (lax.erf(agg * _INV_SQRT2) + 1.0)) * 0.5
        mu = jnp.mean(h, axis=-1, keepdims=True)
        var = jnp.mean((h - mu) * (h - mu), axis=-1, keepdims=True)
        h = (h - mu) * _recip(jnp.sqrt(var + 1e-5)) * g_ref[0] + b_ref[0]
        h = jnp.tanh(jnp.dot(h.astype(jnp.bfloat16), w_ref[...],
                             preferred_element_type=jnp.float32) + d_ref[0])
        if with_resid:
            o_ref[...] = jnp.concatenate([r_ref[...], h], axis=1)
        else:
            o_ref[...] = h

    in_specs = [
        pl.BlockSpec((2, _BN, 128), lambda i: (0, i, 0)),
        pl.BlockSpec((_BN, din), lambda i: (i, 0)),
        pl.BlockSpec((1, din), lambda i: (0, 0)),
        pl.BlockSpec((1, din), lambda i: (0, 0)),
        pl.BlockSpec((din, dout), lambda i: (0, 0)),
        pl.BlockSpec((1, dout), lambda i: (0, 0)),
    ]
    args = [parts, zbw, ln_g, ln_b, wd_t_bf, bd]
    out_cols = dout
    if with_resid:
        in_specs.append(pl.BlockSpec((_BN, resid_in.shape[1]), lambda i: (i, 0)))
        args.append(resid_in)
        out_cols = dout + resid_in.shape[1]

    return pl.pallas_call(
        body,
        grid=(_NB,),
        in_specs=in_specs,
        out_specs=pl.BlockSpec((_BN, out_cols), lambda i: (i, 0)),
        out_shape=jax.ShapeDtypeStruct((NPAD, out_cols), jnp.float32),
    )(*args)


def _stride_perm(x2d, stride):
    return x2d.reshape(EPAD // stride, stride).T.reshape(ROWS_E, 128)


def _layer(state, p, ga, gb, ta, tb, cnt_parts, zeros_init, resid_in=None):
    din = state.shape[1]
    split_cols = din > 128
    bf = jnp.bfloat16
    wts = [jnp.transpose(w).astype(bf) for w in p['W_edge']]
    a_stk = jnp.stack([w[:din] for w in wts])
    b_stk = jnp.stack([w[din:] for w in wts])
    state_bf = state.astype(bf)
    if split_cols:
        a_split = jnp.stack([a_stk[:, :, :128], a_stk[:, :, din - 128:]])
    else:
        a_split = a_stk[None]

    y_flat = _pre_call(state_bf, a_split).reshape(-1, 128)
    zbw = _mix_call(state_bf, b_stk, cnt_parts)
    gidx, tgt2 = (gb, tb) if split_cols else (ga, ta)
    parts = _scatter_call(y_flat, gidx, tgt2, zeros_init, split_cols)
    parts = parts.reshape(2, NPAD, 128)
    return _post_call(parts, zbw, p['ln_g'][None, :], p['ln_b'][None, :],
                      jnp.transpose(p['dense_W']).astype(bf),
                      p['dense_b'][None, :], split_cols, resid_in=resid_in)


def kernel(input_ids, typed_edges, emb_table, params):
    ids = input_ids[0].astype(jnp.int32)
    te = typed_edges[0].astype(jnp.int32)
    etype, src, tgt = te[0], te[1], te[2]

    pe = EPAD - N_EDGES
    pidx = jnp.arange(pe, dtype=jnp.int32)
    et_p = jnp.concatenate([etype, jnp.ones((pe,), jnp.int32)])
    src_p = jnp.concatenate([src, pidx % N_NODES])
    tgt_p = jnp.concatenate([tgt, N_NODES + pidx % (NPAD - N_NODES)])

    g0 = (et_p - 1) * NPAD + src_p
    ga = _stride_perm(g0, NW).astype(jnp.int32)
    gb = jnp.concatenate([_stride_perm(g0, NS),
                          _stride_perm(g0 + T * NPAD, NS)]).astype(jnp.int32)
    ta = _stride_perm(tgt_p, NW).astype(jnp.int32)
    tb = _stride_perm(tgt_p, NS).astype(jnp.int32)

    ids2 = jnp.pad(ids, (0, NPAD - N_NODES)).reshape(NPAD // 128, 128)
    state = _embed_call(emb_table, ids2)

    zeros_init = jnp.zeros((NPAD, 128), jnp.float32)

    lane_t = jnp.arange(128, dtype=jnp.int32) // 16
    onehot = (lane_t[None, :] == jnp.arange(T, dtype=jnp.int32)[:, None])
    y_cnt = jnp.broadcast_to(onehot.astype(jnp.float32)[:, None, :],
                             (T, NPAD, 128)).reshape(T * NPAD, 128)
    cnt_parts = _scatter_call(y_cnt, ga, ta, zeros_init, False)
    cnt_parts = cnt_parts.reshape(2, NPAD, 128)

    for b in ('b0', 'b1'):
        blk = params[b]
        blk_in = state
        for li in range(2):
            state = _layer(state, blk[li], ga, gb, ta, tb, cnt_parts, zeros_init)
        state = _layer(state, blk[2], ga, gb, ta, tb, cnt_parts, zeros_init,
                       resid_in=blk_in)
        state = _layer(state, blk[3], ga, gb, ta, tb, cnt_parts, zeros_init)

    return state[:N_NODES]


# speedup vs baseline: 6.6699x; 1.1404x over previous
import functools

import jax
import jax.numpy as jnp
from jax import lax
from jax.experimental import pallas as pl
from jax.experimental.pallas import tpu as pltpu
from jax.experimental.pallas import tpu_sc as plsc

N_NODES = 10000
N_EDGES = 160000
D = 128
T = 4
NC, NS = 2, 16
NW = NC * NS
NPAD = 10240
EPAD = 163840
ROWS_E = EPAD // 128
_INV_SQRT2 = 0.7071067811865476


def _recip(s):
    r = 1.0 / s
    r = r * (2.0 - s * r)
    r = r * (2.0 - s * r)
    return r

_mesh = plsc.VectorSubcoreMesh(
    core_axis_name="c", subcore_axis_name="s", num_cores=NC, num_subcores=NS
)


def _embed_call(emb_table, ids2):
    @functools.partial(
        pl.kernel,
        out_type=jax.ShapeDtypeStruct((NPAD, D), jnp.float32),
        mesh=_mesh,
        scratch_types=[
            pltpu.VMEM((8, 128), jnp.int32),
            pltpu.VMEM((128, D), jnp.float32),
            pltpu.SemaphoreType.DMA,
        ],
    )
    def k(emb_h, ids_h, out_h, ids_v, rows_v, sem):
        c = lax.axis_index("c")
        s = lax.axis_index("s")
        wid = s * NC + c

        @pl.when(wid < 10)
        def _():
            pltpu.sync_copy(ids_h.at[pl.ds(wid * 8, 8)], ids_v)

            @pl.loop(0, 8)
            def _(j):
                pltpu.async_copy(emb_h.at[ids_v.at[j]], rows_v, sem).wait()
                pltpu.sync_copy(rows_v, out_h.at[pl.ds((wid * 8 + j) * 128, 128)])

    return k(emb_table, ids2)


def _scatter_call(y, gidx, tgt2, zeros_init, split_cols):
    n_super = 10 if split_cols else 5

    @functools.partial(
        pl.kernel,
        out_type=jax.ShapeDtypeStruct((2 * NPAD, 128), jnp.float32),
        mesh=_mesh,
        scratch_types=[
            pltpu.VMEM((8, 128), jnp.int32),
            pltpu.VMEM((8, 128), jnp.int32),
            pltpu.VMEM((128, 128), jnp.float32),
            pltpu.VMEM((128, 128), jnp.float32),
            pltpu.VMEM_SHARED((NPAD, 128), jnp.float32),
            pltpu.SemaphoreType.DMA,
            pltpu.SemaphoreType.DMA,
        ],
    )
    def k(y_h, g_h, t_h, z_h, out_h, idx_v, tgt_v, b0, b1, acc_sh, s0, s1):
        c = lax.axis_index("c")
        s = lax.axis_index("s")
        stripe = NPAD // NS
        pltpu.sync_copy(z_h.at[pl.ds(s * stripe, stripe)],
                        acc_sh.at[pl.ds(s * stripe, stripe)])
        if split_cols:
            idx_base = c * ROWS_E + s * 80
            tgt_base = s * 80
        else:
            wid = s * NC + c
            idx_base = wid * 40
            tgt_base = wid * 40
        plsc.subcore_barrier()

        @pl.loop(0, n_super)
        def _(chunk):
            pltpu.sync_copy(g_h.at[pl.ds(idx_base + chunk * 8, 8)], idx_v)
            pltpu.sync_copy(t_h.at[pl.ds(tgt_base + chunk * 8, 8)], tgt_v)
            pltpu.async_copy(y_h.at[idx_v.at[0]], b0, s0)

            @pl.loop(0, 4)
            def _(kk):
                r = 2 * kk
                pltpu.async_copy(y_h.at[idx_v.at[r + 1]], b1, s1)
                pltpu.make_async_copy(y_h.at[pl.ds(0, 128)], b0, s0).wait()
                pltpu.sync_copy(b0, acc_sh.at[tgt_v.at[r]], add=True)

                @pl.when(r + 2 < 8)
                def _():
                    pltpu.async_copy(y_h.at[idx_v.at[r + 2]], b0, s0)

                pltpu.make_async_copy(y_h.at[pl.ds(0, 128)], b1, s1).wait()
                pltpu.sync_copy(b1, acc_sh.at[tgt_v.at[r + 1]], add=True)

        plsc.subcore_barrier()
        pltpu.sync_copy(acc_sh.at[pl.ds(s * stripe, stripe)],
                        out_h.at[pl.ds(c * NPAD + s * stripe, stripe)])

    return k(y, gidx, tgt2, zeros_init)


_BN = 256
_NB = NPAD // _BN


def _pre_call(state_bf, a_split_bf):
    din = state_bf.shape[1]
    n_copies = a_split_bf.shape[0]

    def body(s_ref, a_ref, y_ref):
        y_ref[0, 0] = jnp.dot(s_ref[...], a_ref[0, 0],
                              preferred_element_type=jnp.float32)

    return pl.pallas_call(
        body,
        grid=(n_copies, T, _NB),
        in_specs=[
            pl.BlockSpec((_BN, din), lambda c, t, i: (i, 0)),
            pl.BlockSpec((1, 1, din, 128), lambda c, t, i: (c, t, 0, 0)),
        ],
        out_specs=pl.BlockSpec((1, 1, _BN, 128), lambda c, t, i: (c, t, i, 0)),
        out_shape=jax.ShapeDtypeStruct((n_copies, T, NPAD, 128), jnp.float32),
    )(state_bf, a_split_bf)


def _mix_call(state_bf, b_stk_bf, cnt_parts):
    din = state_bf.shape[1]

    def body(s_ref, b_ref, c_ref, o_ref):
        sv = s_ref[...]
        cnt = c_ref[0] + c_ref[1]
        acc = jnp.zeros((_BN, din), jnp.float32)
        for t in range(T):
            mm = jnp.dot(sv, b_ref[t], preferred_element_type=jnp.float32)
            acc = acc + cnt[:, 16 * t][:, None] * mm
        o_ref[...] = acc

    return pl.pallas_call(
        body,
        grid=(_NB,),
        in_specs=[
            pl.BlockSpec((_BN, din), lambda i: (i, 0)),
            pl.BlockSpec((T, din, din), lambda i: (0, 0, 0)),
            pl.BlockSpec((2, _BN, 128), lambda i: (0, i, 0)),
        ],
        out_specs=pl.BlockSpec((_BN, din), lambda i: (i, 0)),
        out_shape=jax.ShapeDtypeStruct((NPAD, din), jnp.float32),
    )(state_bf, b_stk_bf, cnt_parts)


def _post_call(parts, zbw, ln_g, ln_b, wd_t_bf, bd, split_cols, resid_in=None):
    din = zbw.shape[1]
    dout = wd_t_bf.shape[1]
    with_resid = resid_in is not None

    def body(*refs):
        if with_resid:
            p_ref, z_ref, g_ref, b_ref, w_ref, d_ref, r_ref, o_ref = refs
        else:
            p_ref, z_ref, g_ref, b_ref, w_ref, d_ref, o_ref = refs
        if split_cols:
            agg = jnp.concatenate([p_ref[0], p_ref[1]], axis=1)
        else:
            agg = p_ref[0] + p_ref[1]
        agg = agg + z_ref[...]
        h = (agg * (lax.erf(agg * _INV_SQRT2) + 1.0)) * 0.5
        mu = jnp.mean(h, axis=-1, keepdims=True)
        var = jnp.mean((h - mu) * (h - mu), axis=-1, keepdims=True)
        h = (h - mu) * _recip(jnp.sqrt(var + 1e-5)) * g_ref[0] + b_ref[0]
        h = jnp.tanh(jnp.dot(h.astype(jnp.bfloat16), w_ref[...],
                             preferred_element_type=jnp.float32) + d_ref[0])
        if with_resid:
            o_ref[...] = jnp.concatenate([r_ref[...], h], axis=1)
        else:
            o_ref[...] = h

    in_specs = [
        pl.BlockSpec((2, _BN, 128), lambda i: (0, i, 0)),
        pl.BlockSpec((_BN, din), lambda i: (i, 0)),
        pl.BlockSpec((1, din), lambda i: (0, 0)),
        pl.BlockSpec((1, din), lambda i: (0, 0)),
        pl.BlockSpec((din, dout), lambda i: (0, 0)),
        pl.BlockSpec((1, dout), lambda i: (0, 0)),
    ]
    args = [parts, zbw, ln_g, ln_b, wd_t_bf, bd]
    out_cols = dout
    if with_resid:
        in_specs.append(pl.BlockSpec((_BN, resid_in.shape[1]), lambda i: (i, 0)))
        args.append(resid_in)
        out_cols = dout + resid_in.shape[1]

    return pl.pallas_call(
        body,
        grid=(_NB,),
        in_specs=in_specs,
        out_specs=pl.BlockSpec((_BN, out_cols), lambda i: (i, 0)),
        out_shape=jax.ShapeDtypeStruct((NPAD, out_cols), jnp.float32),
    )(*args)


def _stride_perm(x2d, stride):
    return x2d.reshape(EPAD // stride, stride).T.reshape(ROWS_E, 128)


def _layer(state, p, ga, gb, ta, tb, cnt_parts, zeros_init, resid_in=None):
    din = state.shape[1]
    split_cols = din > 128
    bf = jnp.bfloat16
    wts = [jnp.transpose(w).astype(bf) for w in p['W_edge']]
    a_stk = jnp.stack([w[:din] for w in wts])
    b_stk = jnp.stack([w[din:] for w in wts])
    state_bf = state.astype(bf)
    if split_cols:
        a_split = jnp.stack([a_stk[:, :, :128], a_stk[:, :, din - 128:]])
    else:
        a_split = a_stk[None]

    y_flat = _pre_call(state_bf, a_split).reshape(-1, 128)
    zbw = _mix_call(state_bf, b_stk, cnt_parts)
    gidx, tgt2 = (gb, tb) if split_cols else (ga, ta)
    parts = _scatter_call(y_flat, gidx, tgt2, zeros_init, split_cols)
    parts = parts.reshape(2, NPAD, 128)
    return _post_call(parts, zbw, p['ln_g'][None, :], p['ln_b'][None, :],
                      jnp.transpose(p['dense_W']).astype(bf),
                      p['dense_b'][None, :], split_cols, resid_in=resid_in)


def kernel(input_ids, typed_edges, emb_table, params):
    ids = input_ids[0].astype(jnp.int32)
    te = typed_edges[0].astype(jnp.int32)
    etype, src, tgt = te[0], te[1], te[2]

    pe = EPAD - N_EDGES
    pidx = jnp.arange(pe, dtype=jnp.int32)
    et_p = jnp.concatenate([etype, jnp.ones((pe,), jnp.int32)])
    src_p = jnp.concatenate([src, pidx % N_NODES])
    tgt_p = jnp.concatenate([tgt, N_NODES + pidx % (NPAD - N_NODES)])

    g0 = (et_p - 1) * NPAD + src_p
    ga = _stride_perm(g0, NW).astype(jnp.int32)
    gb = jnp.concatenate([_stride_perm(g0, NS),
                          _stride_perm(g0 + T * NPAD, NS)]).astype(jnp.int32)
    ta = _stride_perm(tgt_p, NW).astype(jnp.int32)
    tb = _stride_perm(tgt_p, NS).astype(jnp.int32)

    ids2 = jnp.pad(ids, (0, NPAD - N_NODES)).reshape(NPAD // 128, 128)
    state = _embed_call(emb_table, ids2)

    zeros_init = jnp.zeros((NPAD, 128), jnp.float32)

    lane_t = jnp.arange(128, dtype=jnp.int32) // 16
    onehot = (lane_t[None, :] == jnp.arange(T, dtype=jnp.int32)[:, None])
    y_cnt = jnp.broadcast_to(onehot.astype(jnp.float32)[:, None, :],
                             (T, NPAD, 128)).reshape(T * NPAD, 128)
    cnt_parts = _scatter_call(y_cnt, ga, ta, zeros_init, False)
    cnt_parts = cnt_parts.reshape(2, NPAD, 128)

    for b in ('b0', 'b1'):
        blk = params[b]
        blk_in = state
        for li in range(2):
            state = _layer(state, blk[li], ga, gb, ta, tb, cnt_parts, zeros_init)
        state = _layer(state, blk[2], ga, gb, ta, tb, cnt_parts, zeros_init,
                       resid_in=blk_in)
        state = _layer(state, blk[3], ga, gb, ta, tb, cnt_parts, zeros_init)

    return state[:N_NODES]
